# baseline (speedup 1.0000x reference)
import functools

import jax
import jax.numpy as jnp
from jax.experimental import pallas as pl
from jax.experimental.pallas import tpu as pltpu


def _moe_body(x_ref, wa_ref, ba_ref, wb_ref, bb_ref, out_ref, acc2_ref, xbf_ref,
              *, nf):
    f = pl.program_id(1)

    @pl.when(f == 0)
    def _init():
        xbf_ref[:] = x_ref[:].astype(jnp.bfloat16)
        out_ref[:] = jnp.broadcast_to(bb_ref[0:1, :], out_ref.shape)
        acc2_ref[:] = jnp.broadcast_to(bb_ref[1:2, :], acc2_ref.shape)

    xbf = xbf_ref[:]
    h1 = jnp.dot(xbf, wa_ref[0], preferred_element_type=jnp.float32)
    h1 = jnp.maximum(h1 + ba_ref[0:1, :], 0.0).astype(jnp.bfloat16)
    out_ref[:] += jnp.dot(h1, wb_ref[0], preferred_element_type=jnp.float32)

    h2 = jnp.dot(xbf, wa_ref[1], preferred_element_type=jnp.float32)
    h2 = jnp.maximum(h2 + ba_ref[1:2, :], 0.0).astype(jnp.bfloat16)
    acc2_ref[:] += jnp.dot(h2, wb_ref[1], preferred_element_type=jnp.float32)

    @pl.when(f == nf - 1)
    def _select():
        mask = x_ref[:, 0:1] <= 0.0
        out_ref[:] = jnp.where(mask, out_ref[:], acc2_ref[:])


def kernel(x, W1a, b1a, W1b, b1b, W2a, b2a, W2b, b2b):
    n, d = x.shape
    f_dim = W1a.shape[1]
    t = 512 if n % 512 == 0 else n
    ft = 512 if f_dim % 512 == 0 else f_dim
    nt, nf = n // t, f_dim // ft

    wa = jnp.stack([W1a, W2a]).astype(jnp.bfloat16)
    wb = jnp.stack([W1b, W2b]).astype(jnp.bfloat16)
    ba = jnp.stack([b1a, b2a])
    bb = jnp.stack([b1b, b2b])

    grid = (nt, nf)
    out = pl.pallas_call(
        functools.partial(_moe_body, nf=nf),
        grid=grid,
        in_specs=[
            pl.BlockSpec((t, d), lambda i, j: (i, 0)),
            pl.BlockSpec((2, d, ft), lambda i, j: (0, 0, j)),
            pl.BlockSpec((2, ft), lambda i, j: (0, j)),
            pl.BlockSpec((2, ft, d), lambda i, j: (0, j, 0)),
            pl.BlockSpec((2, d), lambda i, j: (0, 0)),
        ],
        out_specs=pl.BlockSpec((t, d), lambda i, j: (i, 0)),
        out_shape=jax.ShapeDtypeStruct((n, d), jnp.float32),
        scratch_shapes=[
            pltpu.VMEM((t, d), jnp.float32),
            pltpu.VMEM((t, d), jnp.bfloat16),
        ],
        compiler_params=pltpu.CompilerParams(
            dimension_semantics=("parallel", "arbitrary"),
        ),
    )(x, wa, ba, wb, bb)
    return out


# speedup vs baseline: 1.3842x; 1.3842x over previous
import dataclasses
import functools

import jax
import jax.numpy as jnp
from jax import lax
from jax.experimental import pallas as pl
from jax.experimental.pallas import tpu as pltpu
from jax.experimental.pallas import tpu_sc as plsc

def _sc_compiler_params():
    cp = pltpu.CompilerParams()
    if "needs_layout_passes" in pltpu.CompilerParams.__dataclass_fields__:
        cp = dataclasses.replace(cp, needs_layout_passes=False)
    return cp


_T = 1024
_FT = 512
_L = 16
_NW = 32
_RC = 32


def _count_masked(xc_ref, lo, hi):
    def body(j, acc):
        v = xc_ref[pl.ds(j * _L, _L)]
        return acc + plsc.all_reduce_population_count(v <= 0.0)
    return lax.fori_loop(lo, hi, body, jnp.zeros((_L,), jnp.int32))


def _route_body(xcol_hbm, x_hbm, invp_hbm, counts_hbm, xs_hbm,
                xc_ref, idx_ref, rows_ref, cnt_ref, sem):
    wid = lax.axis_index("s") * 2 + lax.axis_index("c")
    n = xcol_hbm.shape[0]
    chunk = n // _NW
    nv = chunk // _L
    base = wid * chunk

    pltpu.sync_copy(xcol_hbm, xc_ref)

    c1 = _count_masked(xc_ref, 0, n // _L)
    pre = _count_masked(xc_ref, 0, wid * nv)
    s2 = (c1 + (_T - 1)) & (-_T)

    @pl.when(wid == 0)
    def _():
        cnt_ref[:] = c1
        pltpu.sync_copy(cnt_ref, counts_hbm)

    iota = lax.iota(jnp.int32, _L)
    n1 = pre
    for k in range(nv):
        v = xc_ref[pl.ds((wid * nv + k) * _L, _L)]
        m = v <= 0.0
        mi = jnp.where(m, 1, 0)
        excl1 = n1 + plsc.cumsum(mi) - mi
        pos = base + k * _L + iota
        dest = jnp.where(m, excl1, s2 + (pos - excl1))
        r, half = k // 2, (k % 2) * _L
        idx_ref[r, pl.ds(half, _L)] = dest
        n1 = n1 + plsc.all_reduce_population_count(m)

    pltpu.sync_copy(idx_ref, invp_hbm.at[wid])

    for c in range(chunk // _RC):
        pltpu.sync_copy(x_hbm.at[pl.ds(base + c * _RC, _RC)], rows_ref)
        pltpu.async_copy(rows_ref, xs_hbm.at[idx_ref.at[c]], sem).wait()


def _unpermute_body(outs_hbm, invp_hbm, out_hbm, idx_ref, rows_ref, sem):
    wid = lax.axis_index("s") * 2 + lax.axis_index("c")
    n = out_hbm.shape[0]
    chunk = n // _NW
    base = wid * chunk
    pltpu.sync_copy(invp_hbm.at[wid], idx_ref)
    for c in range(chunk // _RC):
        pltpu.async_copy(outs_hbm.at[idx_ref.at[c]], rows_ref, sem).wait()
        pltpu.sync_copy(rows_ref, out_hbm.at[pl.ds(base + c * _RC, _RC)])


def _moe_body(em_ref, x_ref, wa_ref, ba_ref, wb_ref, bb_ref, out_ref,
              xbf_ref, *, nf):
    f = pl.program_id(1)

    @pl.when(f == 0)
    def _init():
        xbf_ref[:] = x_ref[:].astype(jnp.bfloat16)
        out_ref[:] = jnp.broadcast_to(bb_ref[0], out_ref.shape)

    h = jnp.dot(xbf_ref[:], wa_ref[0], preferred_element_type=jnp.float32)
    h = jnp.maximum(h + ba_ref[0], 0.0).astype(jnp.bfloat16)
    out_ref[:] += jnp.dot(h, wb_ref[0], preferred_element_type=jnp.float32)


def kernel(x, W1a, b1a, W1b, b1b, W2a, b2a, W2b, b2b):
    n, d = x.shape
    f_dim = W1a.shape[1]
    np_ = n + _T
    nt, nf = np_ // _T, f_dim // _FT
    chunk = n // _NW
    mesh = plsc.VectorSubcoreMesh(core_axis_name="c", subcore_axis_name="s")

    route = pl.kernel(
        _route_body,
        mesh=mesh,
        out_type=[
            jax.ShapeDtypeStruct((_NW, chunk // _RC, _RC), jnp.int32),
            jax.ShapeDtypeStruct((_L,), jnp.int32),
            jax.ShapeDtypeStruct((np_, d), jnp.float32),
        ],
        scratch_types=[
            pltpu.VMEM((n,), jnp.float32),
            pltpu.VMEM((chunk // _RC, _RC), jnp.int32),
            pltpu.VMEM((_RC, d), jnp.float32),
            pltpu.VMEM((_L,), jnp.int32),
            pltpu.SemaphoreType.DMA,
        ],
        compiler_params=_sc_compiler_params(),
    )
    inv_perm, counts, x_s = route(x[:, 0], x)

    c1 = counts[0]
    nt1 = (c1 + _T - 1) // _T
    em = (jnp.arange(nt, dtype=jnp.int32) >= nt1).astype(jnp.int32)

    wa = jnp.stack([W1a, W2a]).astype(jnp.bfloat16)
    wb = jnp.stack([W1b, W2b]).astype(jnp.bfloat16)
    ba = jnp.stack([b1a, b2a])[:, None, :]
    bb = jnp.stack([b1b, b2b])[:, None, :]

    out_s = pl.pallas_call(
        functools.partial(_moe_body, nf=nf),
        grid_spec=pltpu.PrefetchScalarGridSpec(
            num_scalar_prefetch=1,
            grid=(nt, nf),
            in_specs=[
                pl.BlockSpec((_T, d), lambda i, j, em: (i, 0)),
                pl.BlockSpec((1, d, _FT), lambda i, j, em: (em[i], 0, j)),
                pl.BlockSpec((1, 1, _FT), lambda i, j, em: (em[i], 0, j)),
                pl.BlockSpec((1, _FT, d), lambda i, j, em: (em[i], j, 0)),
                pl.BlockSpec((1, 1, d), lambda i, j, em: (em[i], 0, 0)),
            ],
            out_specs=pl.BlockSpec((_T, d), lambda i, j, em: (i, 0)),
            scratch_shapes=[pltpu.VMEM((_T, d), jnp.bfloat16)],
        ),
        out_shape=jax.ShapeDtypeStruct((np_, d), jnp.float32),
        compiler_params=pltpu.CompilerParams(
            dimension_semantics=("parallel", "arbitrary"),
        ),
    )(em, x_s, wa, ba, wb, bb)

    unpermute = pl.kernel(
        _unpermute_body,
        mesh=mesh,
        out_type=jax.ShapeDtypeStruct((n, d), jnp.float32),
        scratch_types=[
            pltpu.VMEM((chunk // _RC, _RC), jnp.int32),
            pltpu.VMEM((_RC, d), jnp.float32),
            pltpu.SemaphoreType.DMA,
        ],
        compiler_params=_sc_compiler_params(),
    )
    return unpermute(out_s, inv_perm)


# speedup vs baseline: 1.4275x; 1.0313x over previous
import dataclasses
import functools

import jax
import jax.numpy as jnp
from jax import lax
from jax.experimental import pallas as pl
from jax.experimental.pallas import tpu as pltpu
from jax.experimental.pallas import tpu_sc as plsc

def _sc_compiler_params():
    cp = pltpu.CompilerParams()
    if "needs_layout_passes" in pltpu.CompilerParams.__dataclass_fields__:
        cp = dataclasses.replace(cp, needs_layout_passes=False)
    return cp


_T = 1024
_FT = 1024
_L = 16
_NW = 32
_RC = 32


def _count_masked(xc_ref, lo, hi):
    def body(j, acc):
        v = xc_ref[pl.ds(j * _L, _L)]
        return acc + plsc.all_reduce_population_count(v <= 0.0)
    return lax.fori_loop(lo, hi, body, jnp.zeros((_L,), jnp.int32))


def _route_body(xcol_hbm, x_hbm, invp_hbm, counts_hbm, xs_hbm,
                xc_ref, idx_ref, rows_ref, cnt_ref, sem):
    wid = lax.axis_index("s") * 2 + lax.axis_index("c")
    n = xcol_hbm.shape[0]
    chunk = n // _NW
    nv = chunk // _L
    base = wid * chunk

    pltpu.sync_copy(xcol_hbm, xc_ref)

    c1 = _count_masked(xc_ref, 0, n // _L)
    pre = _count_masked(xc_ref, 0, wid * nv)
    s2 = (c1 + (_T - 1)) & (-_T)

    @pl.when(wid == 0)
    def _():
        cnt_ref[:] = c1
        pltpu.sync_copy(cnt_ref, counts_hbm)

    iota = lax.iota(jnp.int32, _L)
    n1 = pre
    for k in range(nv):
        v = xc_ref[pl.ds((wid * nv + k) * _L, _L)]
        m = v <= 0.0
        mi = jnp.where(m, 1, 0)
        excl1 = n1 + plsc.cumsum(mi) - mi
        pos = base + k * _L + iota
        dest = jnp.where(m, excl1, s2 + (pos - excl1))
        r, half = k // 2, (k % 2) * _L
        idx_ref[r, pl.ds(half, _L)] = dest
        n1 = n1 + plsc.all_reduce_population_count(m)

    pltpu.sync_copy(idx_ref, invp_hbm.at[wid])

    for c in range(chunk // _RC):
        pltpu.sync_copy(x_hbm.at[pl.ds(base + c * _RC, _RC)], rows_ref)
        pltpu.async_copy(rows_ref, xs_hbm.at[idx_ref.at[c]], sem).wait()


def _unpermute_body(outs_hbm, invp_hbm, out_hbm, idx_ref, rows_ref, sem):
    wid = lax.axis_index("s") * 2 + lax.axis_index("c")
    n = out_hbm.shape[0]
    chunk = n // _NW
    base = wid * chunk
    pltpu.sync_copy(invp_hbm.at[wid], idx_ref)
    for c in range(chunk // _RC):
        pltpu.async_copy(outs_hbm.at[idx_ref.at[c]], rows_ref, sem).wait()
        pltpu.sync_copy(rows_ref, out_hbm.at[pl.ds(base + c * _RC, _RC)])


def _moe_body(em_ref, x_ref, wa_ref, ba_ref, wb_ref, bb_ref, out_ref,
              xbf_ref, *, nf):
    f = pl.program_id(1)

    @pl.when(f == 0)
    def _init():
        xbf_ref[:] = x_ref[:].astype(jnp.bfloat16)
        out_ref[:] = jnp.broadcast_to(bb_ref[0], out_ref.shape)

    h = jnp.dot(xbf_ref[:], wa_ref[0], preferred_element_type=jnp.float32)
    h = jnp.maximum(h + ba_ref[0], 0.0).astype(jnp.bfloat16)
    out_ref[:] += jnp.dot(h, wb_ref[0], preferred_element_type=jnp.float32)


def kernel(x, W1a, b1a, W1b, b1b, W2a, b2a, W2b, b2b):
    n, d = x.shape
    f_dim = W1a.shape[1]
    np_ = n + _T
    nt, nf = np_ // _T, f_dim // _FT
    chunk = n // _NW
    mesh = plsc.VectorSubcoreMesh(core_axis_name="c", subcore_axis_name="s")

    route = pl.kernel(
        _route_body,
        mesh=mesh,
        out_type=[
            jax.ShapeDtypeStruct((_NW, chunk // _RC, _RC), jnp.int32),
            jax.ShapeDtypeStruct((_L,), jnp.int32),
            jax.ShapeDtypeStruct((np_, d), jnp.float32),
        ],
        scratch_types=[
            pltpu.VMEM((n,), jnp.float32),
            pltpu.VMEM((chunk // _RC, _RC), jnp.int32),
            pltpu.VMEM((_RC, d), jnp.float32),
            pltpu.VMEM((_L,), jnp.int32),
            pltpu.SemaphoreType.DMA,
        ],
        compiler_params=_sc_compiler_params(),
    )
    inv_perm, counts, x_s = route(x[:, 0], x)

    c1 = counts[0]
    nt1 = (c1 + _T - 1) // _T
    em = (jnp.arange(nt, dtype=jnp.int32) >= nt1).astype(jnp.int32)

    wa = jnp.stack([W1a, W2a]).astype(jnp.bfloat16)
    wb = jnp.stack([W1b, W2b]).astype(jnp.bfloat16)
    ba = jnp.stack([b1a, b2a])[:, None, :]
    bb = jnp.stack([b1b, b2b])[:, None, :]

    out_s = pl.pallas_call(
        functools.partial(_moe_body, nf=nf),
        grid_spec=pltpu.PrefetchScalarGridSpec(
            num_scalar_prefetch=1,
            grid=(nt, nf),
            in_specs=[
                pl.BlockSpec((_T, d), lambda i, j, em: (i, 0)),
                pl.BlockSpec((1, d, _FT), lambda i, j, em: (em[i], 0, j)),
                pl.BlockSpec((1, 1, _FT), lambda i, j, em: (em[i], 0, j)),
                pl.BlockSpec((1, _FT, d), lambda i, j, em: (em[i], j, 0)),
                pl.BlockSpec((1, 1, d), lambda i, j, em: (em[i], 0, 0)),
            ],
            out_specs=pl.BlockSpec((_T, d), lambda i, j, em: (i, 0)),
            scratch_shapes=[pltpu.VMEM((_T, d), jnp.bfloat16)],
        ),
        out_shape=jax.ShapeDtypeStruct((np_, d), jnp.float32),
        compiler_params=pltpu.CompilerParams(
            dimension_semantics=("parallel", "arbitrary"),
        ),
    )(em, x_s, wa, ba, wb, bb)

    unpermute = pl.kernel(
        _unpermute_body,
        mesh=mesh,
        out_type=jax.ShapeDtypeStruct((n, d), jnp.float32),
        scratch_types=[
            pltpu.VMEM((chunk // _RC, _RC), jnp.int32),
            pltpu.VMEM((_RC, d), jnp.float32),
            pltpu.SemaphoreType.DMA,
        ],
        compiler_params=_sc_compiler_params(),
    )
    return unpermute(out_s, inv_perm)


# speedup vs baseline: 1.6035x; 1.1233x over previous
import dataclasses
import functools

import jax
import jax.numpy as jnp
from jax import lax
from jax.experimental import pallas as pl
from jax.experimental.pallas import tpu as pltpu
from jax.experimental.pallas import tpu_sc as plsc

def _sc_compiler_params():
    cp = pltpu.CompilerParams()
    if "needs_layout_passes" in pltpu.CompilerParams.__dataclass_fields__:
        cp = dataclasses.replace(cp, needs_layout_passes=False)
    return cp


_T = 1024
_FT = 512
_L = 16
_NW = 32
_RC = 32


def _count_masked(xc_ref, lo, hi):
    def body(j, acc):
        v = xc_ref[pl.ds(j * _L, _L)]
        return acc + plsc.all_reduce_population_count(v <= 0.0)
    return lax.fori_loop(lo, hi, body, jnp.zeros((_L,), jnp.int32))


def _route_body(xcol_hbm, x_hbm, invp_hbm, counts_hbm, xs_hbm,
                xc_ref, idx_ref, rows_ref, cnt_ref, sem):
    wid = lax.axis_index("s") * 2 + lax.axis_index("c")
    n = xcol_hbm.shape[0]
    chunk = n // _NW
    nv = chunk // _L
    base = wid * chunk

    pltpu.sync_copy(xcol_hbm, xc_ref)

    c1 = _count_masked(xc_ref, 0, n // _L)
    pre = _count_masked(xc_ref, 0, wid * nv)
    s2 = (c1 + (_T - 1)) & (-_T)

    @pl.when(wid == 0)
    def _():
        cnt_ref[:] = c1
        pltpu.sync_copy(cnt_ref, counts_hbm)

    iota = lax.iota(jnp.int32, _L)
    n1 = pre
    for k in range(nv):
        v = xc_ref[pl.ds((wid * nv + k) * _L, _L)]
        m = v <= 0.0
        mi = jnp.where(m, 1, 0)
        excl1 = n1 + plsc.cumsum(mi) - mi
        pos = base + k * _L + iota
        dest = jnp.where(m, excl1, s2 + (pos - excl1))
        r, half = k // 2, (k % 2) * _L
        idx_ref[r, pl.ds(half, _L)] = dest
        n1 = n1 + plsc.all_reduce_population_count(m)

    pltpu.sync_copy(idx_ref, invp_hbm.at[wid])

    for c in range(chunk // _RC):
        pltpu.sync_copy(x_hbm.at[pl.ds(base + c * _RC, _RC)], rows_ref)
        pltpu.async_copy(rows_ref, xs_hbm.at[idx_ref.at[c]], sem).wait()


def _unpermute_body(outs_hbm, invp_hbm, out_hbm, idx_ref, rows_ref, sem):
    wid = lax.axis_index("s") * 2 + lax.axis_index("c")
    n = out_hbm.shape[0]
    chunk = n // _NW
    base = wid * chunk
    pltpu.sync_copy(invp_hbm.at[wid], idx_ref)
    for c in range(chunk // _RC):
        pltpu.async_copy(outs_hbm.at[idx_ref.at[c]], rows_ref, sem).wait()
        pltpu.sync_copy(rows_ref, out_hbm.at[pl.ds(base + c * _RC, _RC)])


def _moe_body(em_ref, x_ref, wa1_ref, ba1_ref, wb1_ref, bb1_ref,
              wa2_ref, ba2_ref, wb2_ref, bb2_ref, out_ref, *, nf):
    t, f = pl.program_id(0), pl.program_id(1)

    def expert(wa_ref, ba_ref, wb_ref, bb_ref):
        @pl.when(f == 0)
        def _init():
            out_ref[:] = jnp.broadcast_to(bb_ref[0:1, :], out_ref.shape)

        h = jnp.dot(x_ref[:], wa_ref[:], preferred_element_type=jnp.float32)
        h = jnp.maximum(h + ba_ref[0:1, :], 0.0).astype(jnp.bfloat16)
        out_ref[:] += jnp.dot(h, wb_ref[:], preferred_element_type=jnp.float32)

    @pl.when(em_ref[t] == 0)
    def _e1():
        expert(wa1_ref, ba1_ref, wb1_ref, bb1_ref)

    @pl.when(em_ref[t] != 0)
    def _e2():
        expert(wa2_ref, ba2_ref, wb2_ref, bb2_ref)


def kernel(x, W1a, b1a, W1b, b1b, W2a, b2a, W2b, b2b):
    n, d = x.shape
    f_dim = W1a.shape[1]
    np_ = n + _T
    nt, nf = np_ // _T, f_dim // _FT
    chunk = n // _NW
    mesh = plsc.VectorSubcoreMesh(core_axis_name="c", subcore_axis_name="s")

    route = pl.kernel(
        _route_body,
        mesh=mesh,
        out_type=[
            jax.ShapeDtypeStruct((_NW, chunk // _RC, _RC), jnp.int32),
            jax.ShapeDtypeStruct((_L,), jnp.int32),
            jax.ShapeDtypeStruct((np_, d), jnp.float32),
        ],
        scratch_types=[
            pltpu.VMEM((n,), jnp.float32),
            pltpu.VMEM((chunk // _RC, _RC), jnp.int32),
            pltpu.VMEM((_RC, d), jnp.float32),
            pltpu.VMEM((_L,), jnp.int32),
            pltpu.SemaphoreType.DMA,
        ],
        compiler_params=_sc_compiler_params(),
    )
    inv_perm, counts, x_s = route(x[:, 0], x)

    c1 = counts[0]
    nt1 = (c1 + _T - 1) // _T
    em = (jnp.arange(nt, dtype=jnp.int32) >= nt1).astype(jnp.int32)

    def wsel(e):
        def fa(i, j, em):
            return (0, jnp.where(em[i] == e, j, 0))

        def fb(i, j, em):
            return (jnp.where(em[i] == e, j, 0), 0)

        def fbias(i, j, em):
            return (0, jnp.where(em[i] == e, j, 0))

        return (
            pl.BlockSpec((d, _FT), fa),
            pl.BlockSpec((1, _FT), fbias),
            pl.BlockSpec((_FT, d), fb),
            pl.BlockSpec((1, d), lambda i, j, em: (0, 0)),
        )

    out_s = pl.pallas_call(
        functools.partial(_moe_body, nf=nf),
        grid_spec=pltpu.PrefetchScalarGridSpec(
            num_scalar_prefetch=1,
            grid=(nt, nf),
            in_specs=[
                pl.BlockSpec((_T, d), lambda i, j, em: (i, 0)),
                *wsel(0), *wsel(1),
            ],
            out_specs=pl.BlockSpec((_T, d), lambda i, j, em: (i, 0)),
        ),
        out_shape=jax.ShapeDtypeStruct((np_, d), jnp.float32),
        compiler_params=pltpu.CompilerParams(
            dimension_semantics=("parallel", "arbitrary"),
        ),
    )(em, x_s,
      W1a, b1a[None, :], W1b.astype(jnp.bfloat16), b1b[None, :],
      W2a, b2a[None, :], W2b.astype(jnp.bfloat16), b2b[None, :])

    unpermute = pl.kernel(
        _unpermute_body,
        mesh=mesh,
        out_type=jax.ShapeDtypeStruct((n, d), jnp.float32),
        scratch_types=[
            pltpu.VMEM((chunk // _RC, _RC), jnp.int32),
            pltpu.VMEM((_RC, d), jnp.float32),
            pltpu.SemaphoreType.DMA,
        ],
        compiler_params=_sc_compiler_params(),
    )
    return unpermute(out_s, inv_perm)
